# baseline (speedup 1.0000x reference)
import functools
import math

import jax
import jax.numpy as jnp
from jax.experimental import pallas as pl
from jax.experimental.pallas import tpu as pltpu

B, T, N = 16, 24, 256
D = 128
M = 2048
K = 8
H = 4
HD = D // H
T_OUT = 25
KW = 12
TAU_TIME = 2.0

M_BLK = 64
O_BLK = 2048


def _gelu(x):
    return 0.5 * x * (1.0 + jax.lax.erf(x * (1.0 / math.sqrt(2.0))))


def _conv_kernel(x_ref, w_ref, b_ref, out_ref):
    x = x_ref[...]
    acc = jnp.zeros((B, T_OUT, N), jnp.float32)
    for j in range(KW):
        acc = acc + x[:, j:j + T_OUT, :] * w_ref[j]
    acc = _gelu(acc + b_ref[...])
    out_ref[...] = jnp.transpose(acc, (1, 0, 2))


def _pw_kernel(h_ref, w_ref, b_ref, out_ref):
    p = jax.lax.dot_general(h_ref[...], w_ref[...], (((1,), (1,)), ((), ())),
                            preferred_element_type=jnp.float32)
    p = _gelu(p + b_ref[...])
    acc = jnp.zeros((B, O_BLK), jnp.float32)
    for t in range(T_OUT):
        acc = acc + p[t * B:(t + 1) * B, :]
    out_ref[...] = acc * (1.0 / T_OUT)


def _ln_kernel(x_ref, w_ref, b_ref, out_ref):
    x = x_ref[...]
    mu = jnp.mean(x, axis=-1, keepdims=True)
    var = jnp.mean((x - mu) ** 2, axis=-1, keepdims=True)
    out_ref[...] = (x - mu) / jnp.sqrt(var + 1e-5) * w_ref[...] + b_ref[...]


def _sim_kernel(q_ref, mem_ref, msn_ref, myr_ref, sq_ref, yq_ref, out_ref,
                qn_scr):
    @pl.when(pl.program_id(0) == 0)
    def _():
        q = q_ref[...].reshape(B, N * D)
        qsq = jnp.sum(q * q, axis=1, keepdims=True)
        qn_scr[...] = q / jnp.maximum(jnp.sqrt(qsq), 1e-12)

    mb = mem_ref[...].reshape(M_BLK, N * D)
    nsq = jnp.sum(mb * mb, axis=1, keepdims=True)
    mbn = mb / jnp.maximum(jnp.sqrt(nsq), 1e-12)
    sim = jax.lax.dot_general(mbn, qn_scr[...], (((1,), (1,)), ((), ())),
                              preferred_element_type=jnp.float32)
    mask = msn_ref[...] == sq_ref[...]
    sim = jnp.where(mask, sim, -10000.0)
    delta = jnp.abs(myr_ref[...] - yq_ref[...])
    div = 1.0 - jnp.exp(-delta / TAU_TIME)
    out_ref[...] = sim * (0.5 + 0.5 * div)


def _topk_kernel(sim_ref, out_ref):
    work = sim_ref[...]
    iota = jax.lax.broadcasted_iota(jnp.int32, (M, B), 0)
    rows = []
    for _ in range(K):
        mx = jnp.max(work, axis=0, keepdims=True)
        hit = work == mx
        idx = jnp.min(jnp.where(hit, iota, M), axis=0, keepdims=True)
        rows.append(idx)
        work = jnp.where(iota == idx, -jnp.inf, work)
    out_ref[...] = jnp.concatenate(rows, axis=0)


def _attn_kernel(idx_ref, q_ref, *refs):
    mem_refs = refs[:K]
    wq, wk, wv, bq, bk, bv, ow, ob, pw, pb, out_ref = refs[K:]
    gi = jax.lax.broadcasted_iota(jnp.int32, (D, H), 0)
    gj = jax.lax.broadcasted_iota(jnp.int32, (D, H), 1)
    g = (gi // HD == gj).astype(jnp.float32)
    gt = (jax.lax.broadcasted_iota(jnp.int32, (H, D), 1) // HD ==
          jax.lax.broadcasted_iota(jnp.int32, (H, D), 0)).astype(jnp.float32)

    qp = jnp.dot(q_ref[0], wq[...],
                 preferred_element_type=jnp.float32) + bq[...]
    r_all = jnp.concatenate([m[0] for m in mem_refs], axis=0)
    kp = jnp.dot(r_all, wk[...],
                 preferred_element_type=jnp.float32) + bk[...]
    vp = jnp.dot(r_all, wv[...],
                 preferred_element_type=jnp.float32) + bv[...]
    qp_t = jnp.concatenate([qp] * K, axis=0)
    prod = qp_t * kp * (1.0 / math.sqrt(HD))
    logits = jnp.dot(prod, g,
                     preferred_element_type=jnp.float32)
    logits = logits.reshape(K, N, H)
    mx = jnp.max(logits, axis=0, keepdims=True)
    e = jnp.exp(logits - mx)
    att = e / jnp.sum(e, axis=0, keepdims=True)
    att_exp = jnp.dot(att.reshape(K * N, H), gt,
                      preferred_element_type=jnp.float32)
    o = jnp.sum((att_exp * vp).reshape(K, N, D), axis=0)
    attn = jnp.dot(o, ow[...], preferred_element_type=jnp.float32) + ob[...]
    out_ref[0] = jnp.dot(attn, pw[...],
                         preferred_element_type=jnp.float32) + pb[...]


def kernel(x_scalar, season_q, year_q, dw_w, dw_b, pw_w, pw_b, ln_w, ln_b,
           in_proj_w, in_proj_b, out_proj_w, out_proj_b, proj_w, proj_b,
           memory_bank, memory_seasons, memory_years):
    f32 = jnp.float32
    x_scalar = x_scalar.astype(f32)
    season_q = season_q.astype(jnp.int32)
    year_q = year_q.astype(f32)
    memory_seasons = memory_seasons.astype(jnp.int32)
    memory_years = memory_years.astype(f32)

    x_pad = jnp.pad(x_scalar, ((0, 0), (6, 6), (0, 0)))
    w_t = jnp.transpose(dw_w[:, 0, :]).reshape(KW, 1, N)
    h1 = pl.pallas_call(
        _conv_kernel,
        out_shape=jax.ShapeDtypeStruct((T_OUT, B, N), f32),
    )(x_pad, w_t, dw_b.reshape(1, 1, N))
    h1 = h1.reshape(T_OUT * B, N)

    n_o = (N * D) // O_BLK
    q_pre = pl.pallas_call(
        _pw_kernel,
        grid=(n_o,),
        in_specs=[
            pl.BlockSpec((T_OUT * B, N), lambda o: (0, 0)),
            pl.BlockSpec((O_BLK, N), lambda o: (o, 0)),
            pl.BlockSpec((1, O_BLK), lambda o: (0, o)),
        ],
        out_specs=pl.BlockSpec((B, O_BLK), lambda o: (0, o)),
        out_shape=jax.ShapeDtypeStruct((B, N * D), f32),
    )(h1, pw_w, pw_b.reshape(1, N * D))

    q = pl.pallas_call(
        _ln_kernel,
        out_shape=jax.ShapeDtypeStruct((B * N, D), f32),
    )(q_pre.reshape(B * N, D), ln_w.reshape(1, D), ln_b.reshape(1, D))
    q3 = q.reshape(B, N, D)
    q_flat = q.reshape(B, N * D)

    n_m = M // M_BLK
    sim_t = pl.pallas_call(
        _sim_kernel,
        grid=(n_m,),
        in_specs=[
            pl.BlockSpec((B, N, D), lambda m: (0, 0, 0)),
            pl.BlockSpec((M_BLK, N, D), lambda m: (m, 0, 0)),
            pl.BlockSpec((M_BLK, 1), lambda m: (m, 0)),
            pl.BlockSpec((M_BLK, 1), lambda m: (m, 0)),
            pl.BlockSpec((1, B), lambda m: (0, 0)),
            pl.BlockSpec((1, B), lambda m: (0, 0)),
        ],
        out_specs=pl.BlockSpec((M_BLK, B), lambda m: (m, 0)),
        out_shape=jax.ShapeDtypeStruct((M, B), f32),
        scratch_shapes=[pltpu.VMEM((B, N * D), f32)],
    )(q3, memory_bank, memory_seasons.reshape(M, 1),
      memory_years.reshape(M, 1), season_q.reshape(1, B), year_q.reshape(1, B))

    topk_idx = pl.pallas_call(
        _topk_kernel,
        out_shape=jax.ShapeDtypeStruct((K, B), jnp.int32),
    )(sim_t)

    wq_t = jnp.transpose(in_proj_w[:D])
    wk_t = jnp.transpose(in_proj_w[D:2 * D])
    wv_t = jnp.transpose(in_proj_w[2 * D:])
    bq = in_proj_b[:D].reshape(1, D)
    bk = in_proj_b[D:2 * D].reshape(1, D)
    bv = in_proj_b[2 * D:].reshape(1, D)
    ow_t = jnp.transpose(out_proj_w)
    pw_t = jnp.transpose(proj_w)

    mem_specs = [
        pl.BlockSpec((1, N, D), functools.partial(
            lambda b, idx, kk: (idx[kk, b], 0, 0), kk=k))
        for k in range(K)
    ]
    out = pl.pallas_call(
        _attn_kernel,
        grid_spec=pltpu.PrefetchScalarGridSpec(
            num_scalar_prefetch=1,
            grid=(B,),
            in_specs=[pl.BlockSpec((1, N, D), lambda b, idx: (b, 0, 0))]
            + mem_specs
            + [
                pl.BlockSpec((D, D), lambda b, idx: (0, 0)),
                pl.BlockSpec((D, D), lambda b, idx: (0, 0)),
                pl.BlockSpec((D, D), lambda b, idx: (0, 0)),
                pl.BlockSpec((1, D), lambda b, idx: (0, 0)),
                pl.BlockSpec((1, D), lambda b, idx: (0, 0)),
                pl.BlockSpec((1, D), lambda b, idx: (0, 0)),
                pl.BlockSpec((D, D), lambda b, idx: (0, 0)),
                pl.BlockSpec((1, D), lambda b, idx: (0, 0)),
                pl.BlockSpec((D, D), lambda b, idx: (0, 0)),
                pl.BlockSpec((1, D), lambda b, idx: (0, 0)),
            ],
            out_specs=pl.BlockSpec((1, N, D), lambda b, idx: (b, 0, 0)),
        ),
        out_shape=jax.ShapeDtypeStruct((B, N, D), f32),
    )(topk_idx, q3, *([memory_bank] * K), wq_t, wk_t, wv_t, bq, bk, bv,
      ow_t, out_proj_b.reshape(1, D), pw_t, proj_b.reshape(1, D))

    return (out, q3)


# speedup vs baseline: 1.0062x; 1.0062x over previous
import functools
import math

import jax
import jax.numpy as jnp
from jax.experimental import pallas as pl
from jax.experimental.pallas import tpu as pltpu

B, T, N = 16, 24, 256
D = 128
M = 2048
K = 8
H = 4
HD = D // H
T_OUT = 25
KW = 12
TAU_TIME = 2.0

M_BLK = 128
O_BLK = 2048


def _gelu(x):
    return 0.5 * x * (1.0 + jax.lax.erf(x * (1.0 / math.sqrt(2.0))))


def _conv_kernel(x_ref, w_ref, b_ref, out_ref):
    x = x_ref[...]
    acc = jnp.zeros((B, T_OUT, N), jnp.float32)
    for j in range(KW):
        acc = acc + x[:, j:j + T_OUT, :] * w_ref[j]
    acc = _gelu(acc + b_ref[...])
    out_ref[...] = jnp.transpose(acc, (1, 0, 2))


def _pw_kernel(h_ref, w_ref, b_ref, out_ref):
    p = jax.lax.dot_general(h_ref[...], w_ref[...], (((1,), (1,)), ((), ())),
                            preferred_element_type=jnp.float32)
    p = _gelu(p + b_ref[...])
    acc = jnp.zeros((B, O_BLK), jnp.float32)
    for t in range(T_OUT):
        acc = acc + p[t * B:(t + 1) * B, :]
    out_ref[...] = acc * (1.0 / T_OUT)


def _ln_kernel(x_ref, w_ref, b_ref, out_ref):
    x = x_ref[...]
    mu = jnp.mean(x, axis=-1, keepdims=True)
    var = jnp.mean((x - mu) ** 2, axis=-1, keepdims=True)
    out_ref[...] = (x - mu) / jnp.sqrt(var + 1e-5) * w_ref[...] + b_ref[...]


def _sim_kernel(q_ref, mem_ref, msn_ref, myr_ref, sq_ref, yq_ref, out_ref,
                qn_scr):
    @pl.when(pl.program_id(0) == 0)
    def _():
        q = q_ref[...].reshape(B, N * D)
        qsq = jnp.sum(q * q, axis=1, keepdims=True)
        qn_scr[...] = q / jnp.maximum(jnp.sqrt(qsq), 1e-12)

    mb = mem_ref[...].reshape(M_BLK, N * D)
    nsq = jnp.sum(mb * mb, axis=1, keepdims=True)
    mbn = mb / jnp.maximum(jnp.sqrt(nsq), 1e-12)
    sim = jax.lax.dot_general(mbn, qn_scr[...], (((1,), (1,)), ((), ())),
                              preferred_element_type=jnp.float32)
    mask = msn_ref[...] == sq_ref[...]
    sim = jnp.where(mask, sim, -10000.0)
    delta = jnp.abs(myr_ref[...] - yq_ref[...])
    div = 1.0 - jnp.exp(-delta / TAU_TIME)
    out_ref[...] = sim * (0.5 + 0.5 * div)


def _topk_kernel(sim_ref, out_ref):
    work = sim_ref[...]
    iota = jax.lax.broadcasted_iota(jnp.int32, (M, B), 0)
    rows = []
    for _ in range(K):
        mx = jnp.max(work, axis=0, keepdims=True)
        hit = work == mx
        idx = jnp.min(jnp.where(hit, iota, M), axis=0, keepdims=True)
        rows.append(idx)
        work = jnp.where(iota == idx, -jnp.inf, work)
    out_ref[...] = jnp.concatenate(rows, axis=0)


def _attn_kernel(idx_ref, q_ref, *refs):
    mem_refs = refs[:K]
    wq, wk, wv, bq, bk, bv, ow, ob, pw, pb, out_ref = refs[K:]
    gi = jax.lax.broadcasted_iota(jnp.int32, (D, H), 0)
    gj = jax.lax.broadcasted_iota(jnp.int32, (D, H), 1)
    g = (gi // HD == gj).astype(jnp.float32)
    gt = (jax.lax.broadcasted_iota(jnp.int32, (H, D), 1) // HD ==
          jax.lax.broadcasted_iota(jnp.int32, (H, D), 0)).astype(jnp.float32)

    qp = jnp.dot(q_ref[0], wq[...],
                 preferred_element_type=jnp.float32) + bq[...]
    r_all = jnp.concatenate([m[0] for m in mem_refs], axis=0)
    kp = jnp.dot(r_all, wk[...],
                 preferred_element_type=jnp.float32) + bk[...]
    vp = jnp.dot(r_all, wv[...],
                 preferred_element_type=jnp.float32) + bv[...]
    qp_t = jnp.concatenate([qp] * K, axis=0)
    prod = qp_t * kp * (1.0 / math.sqrt(HD))
    logits = jnp.dot(prod, g,
                     preferred_element_type=jnp.float32)
    logits = logits.reshape(K, N, H)
    mx = jnp.max(logits, axis=0, keepdims=True)
    e = jnp.exp(logits - mx)
    att = e / jnp.sum(e, axis=0, keepdims=True)
    att_exp = jnp.dot(att.reshape(K * N, H), gt,
                      preferred_element_type=jnp.float32)
    o = jnp.sum((att_exp * vp).reshape(K, N, D), axis=0)
    attn = jnp.dot(o, ow[...], preferred_element_type=jnp.float32) + ob[...]
    out_ref[0] = jnp.dot(attn, pw[...],
                         preferred_element_type=jnp.float32) + pb[...]


def kernel(x_scalar, season_q, year_q, dw_w, dw_b, pw_w, pw_b, ln_w, ln_b,
           in_proj_w, in_proj_b, out_proj_w, out_proj_b, proj_w, proj_b,
           memory_bank, memory_seasons, memory_years):
    f32 = jnp.float32
    x_scalar = x_scalar.astype(f32)
    season_q = season_q.astype(jnp.int32)
    year_q = year_q.astype(f32)
    memory_seasons = memory_seasons.astype(jnp.int32)
    memory_years = memory_years.astype(f32)

    x_pad = jnp.pad(x_scalar, ((0, 0), (6, 6), (0, 0)))
    w_t = jnp.transpose(dw_w[:, 0, :]).reshape(KW, 1, N)
    h1 = pl.pallas_call(
        _conv_kernel,
        out_shape=jax.ShapeDtypeStruct((T_OUT, B, N), f32),
    )(x_pad, w_t, dw_b.reshape(1, 1, N))
    h1 = h1.reshape(T_OUT * B, N)

    n_o = (N * D) // O_BLK
    q_pre = pl.pallas_call(
        _pw_kernel,
        grid=(n_o,),
        in_specs=[
            pl.BlockSpec((T_OUT * B, N), lambda o: (0, 0)),
            pl.BlockSpec((O_BLK, N), lambda o: (o, 0)),
            pl.BlockSpec((1, O_BLK), lambda o: (0, o)),
        ],
        out_specs=pl.BlockSpec((B, O_BLK), lambda o: (0, o)),
        out_shape=jax.ShapeDtypeStruct((B, N * D), f32),
    )(h1, pw_w, pw_b.reshape(1, N * D))

    q = pl.pallas_call(
        _ln_kernel,
        out_shape=jax.ShapeDtypeStruct((B * N, D), f32),
    )(q_pre.reshape(B * N, D), ln_w.reshape(1, D), ln_b.reshape(1, D))
    q3 = q.reshape(B, N, D)
    q_flat = q.reshape(B, N * D)

    n_m = M // M_BLK
    sim_t = pl.pallas_call(
        _sim_kernel,
        grid=(n_m,),
        in_specs=[
            pl.BlockSpec((B, N, D), lambda m: (0, 0, 0)),
            pl.BlockSpec((M_BLK, N, D), lambda m: (m, 0, 0)),
            pl.BlockSpec((M_BLK, 1), lambda m: (m, 0)),
            pl.BlockSpec((M_BLK, 1), lambda m: (m, 0)),
            pl.BlockSpec((1, B), lambda m: (0, 0)),
            pl.BlockSpec((1, B), lambda m: (0, 0)),
        ],
        out_specs=pl.BlockSpec((M_BLK, B), lambda m: (m, 0)),
        out_shape=jax.ShapeDtypeStruct((M, B), f32),
        scratch_shapes=[pltpu.VMEM((B, N * D), f32)],
    )(q3, memory_bank, memory_seasons.reshape(M, 1),
      memory_years.reshape(M, 1), season_q.reshape(1, B), year_q.reshape(1, B))

    topk_idx = pl.pallas_call(
        _topk_kernel,
        out_shape=jax.ShapeDtypeStruct((K, B), jnp.int32),
    )(sim_t)

    wq_t = jnp.transpose(in_proj_w[:D])
    wk_t = jnp.transpose(in_proj_w[D:2 * D])
    wv_t = jnp.transpose(in_proj_w[2 * D:])
    bq = in_proj_b[:D].reshape(1, D)
    bk = in_proj_b[D:2 * D].reshape(1, D)
    bv = in_proj_b[2 * D:].reshape(1, D)
    ow_t = jnp.transpose(out_proj_w)
    pw_t = jnp.transpose(proj_w)

    mem_specs = [
        pl.BlockSpec((1, N, D), functools.partial(
            lambda b, idx, kk: (idx[kk, b], 0, 0), kk=k))
        for k in range(K)
    ]
    out = pl.pallas_call(
        _attn_kernel,
        grid_spec=pltpu.PrefetchScalarGridSpec(
            num_scalar_prefetch=1,
            grid=(B,),
            in_specs=[pl.BlockSpec((1, N, D), lambda b, idx: (b, 0, 0))]
            + mem_specs
            + [
                pl.BlockSpec((D, D), lambda b, idx: (0, 0)),
                pl.BlockSpec((D, D), lambda b, idx: (0, 0)),
                pl.BlockSpec((D, D), lambda b, idx: (0, 0)),
                pl.BlockSpec((1, D), lambda b, idx: (0, 0)),
                pl.BlockSpec((1, D), lambda b, idx: (0, 0)),
                pl.BlockSpec((1, D), lambda b, idx: (0, 0)),
                pl.BlockSpec((D, D), lambda b, idx: (0, 0)),
                pl.BlockSpec((1, D), lambda b, idx: (0, 0)),
                pl.BlockSpec((D, D), lambda b, idx: (0, 0)),
                pl.BlockSpec((1, D), lambda b, idx: (0, 0)),
            ],
            out_specs=pl.BlockSpec((1, N, D), lambda b, idx: (b, 0, 0)),
        ),
        out_shape=jax.ShapeDtypeStruct((B, N, D), f32),
    )(topk_idx, q3, *([memory_bank] * K), wq_t, wk_t, wv_t, bq, bk, bv,
      ow_t, out_proj_b.reshape(1, D), pw_t, proj_b.reshape(1, D))

    return (out, q3)


# speedup vs baseline: 1.0416x; 1.0351x over previous
import functools
import math

import jax
import jax.numpy as jnp
from jax.experimental import pallas as pl
from jax.experimental.pallas import tpu as pltpu

B, T, N = 16, 24, 256
D = 128
M = 2048
K = 8
H = 4
HD = D // H
T_OUT = 25
KW = 12
TAU_TIME = 2.0

M_BLK = 128
O_BLK = 2048


def _gelu(x):
    return 0.5 * x * (1.0 + jax.lax.erf(x * (1.0 / math.sqrt(2.0))))


def _enc_kernel(x_ref, cw_ref, cb_ref, w_ref, b_ref, out_ref, h_scr):
    @pl.when(pl.program_id(0) == 0)
    def _():
        x = x_ref[...]
        acc = jnp.zeros((B, T_OUT, N), jnp.float32)
        for j in range(KW):
            acc = acc + x[:, j:j + T_OUT, :] * cw_ref[j]
        acc = _gelu(acc + cb_ref[...])
        h_scr[...] = jnp.transpose(acc, (1, 0, 2)).reshape(T_OUT * B, N)

    p = jax.lax.dot_general(h_scr[...], w_ref[...], (((1,), (1,)), ((), ())),
                            preferred_element_type=jnp.float32)
    p = _gelu(p + b_ref[...])
    acc = jnp.zeros((B, O_BLK), jnp.float32)
    for t in range(T_OUT):
        acc = acc + p[t * B:(t + 1) * B, :]
    out_ref[...] = acc * (1.0 / T_OUT)


def _simtop_kernel(qp_ref, mem_ref, msn_ref, myr_ref, sq_ref, yq_ref,
                   lnw_ref, lnb_ref, q_out, idx_out, qn_scr, sim_scr, *, n_m):
    m = pl.program_id(0)

    @pl.when(m == 0)
    def _():
        qp = qp_ref[...]
        mu = jnp.mean(qp, axis=-1, keepdims=True)
        var = jnp.mean((qp - mu) ** 2, axis=-1, keepdims=True)
        qln = (qp - mu) / jnp.sqrt(var + 1e-5) * lnw_ref[...] + lnb_ref[...]
        q_out[...] = qln
        qf = qln.reshape(B, N * D)
        qsq = jnp.sum(qf * qf, axis=1, keepdims=True)
        qn_scr[...] = qf / jnp.maximum(jnp.sqrt(qsq), 1e-12)

    mb = mem_ref[...].reshape(M_BLK, N * D)
    nsq = jnp.sum(mb * mb, axis=1, keepdims=True)
    mbn = mb / jnp.maximum(jnp.sqrt(nsq), 1e-12)
    sim = jax.lax.dot_general(mbn, qn_scr[...], (((1,), (1,)), ((), ())),
                              preferred_element_type=jnp.float32)
    mask = msn_ref[...] == sq_ref[...]
    sim = jnp.where(mask, sim, -10000.0)
    delta = jnp.abs(myr_ref[...] - yq_ref[...])
    div = 1.0 - jnp.exp(-delta / TAU_TIME)
    sim_scr[pl.ds(m * M_BLK, M_BLK), :] = sim * (0.5 + 0.5 * div)

    @pl.when(m == n_m - 1)
    def _():
        work = sim_scr[...]
        iota = jax.lax.broadcasted_iota(jnp.int32, (M, B), 0)
        rows = []
        for _ in range(K):
            mx = jnp.max(work, axis=0, keepdims=True)
            hit = work == mx
            idx = jnp.min(jnp.where(hit, iota, M), axis=0, keepdims=True)
            rows.append(idx)
            work = jnp.where(iota == idx, -jnp.inf, work)
        idx_out[...] = jnp.concatenate(rows, axis=0)


def _attn_kernel(idx_ref, q_ref, *refs):
    mem_refs = refs[:K]
    wq, wk, wv, bq, bk, bv, ow, ob, pw, pb, out_ref = refs[K:]
    gi = jax.lax.broadcasted_iota(jnp.int32, (D, H), 0)
    gj = jax.lax.broadcasted_iota(jnp.int32, (D, H), 1)
    g = (gi // HD == gj).astype(jnp.float32)
    gt = (jax.lax.broadcasted_iota(jnp.int32, (H, D), 1) // HD ==
          jax.lax.broadcasted_iota(jnp.int32, (H, D), 0)).astype(jnp.float32)

    qp = jnp.dot(q_ref[0], wq[...],
                 preferred_element_type=jnp.float32) + bq[...]
    r_all = jnp.concatenate([m[0] for m in mem_refs], axis=0)
    kp = jnp.dot(r_all, wk[...],
                 preferred_element_type=jnp.float32) + bk[...]
    vp = jnp.dot(r_all, wv[...],
                 preferred_element_type=jnp.float32) + bv[...]
    qp_t = jnp.concatenate([qp] * K, axis=0)
    prod = qp_t * kp * (1.0 / math.sqrt(HD))
    logits = jnp.dot(prod, g,
                     preferred_element_type=jnp.float32)
    logits = logits.reshape(K, N, H)
    mx = jnp.max(logits, axis=0, keepdims=True)
    e = jnp.exp(logits - mx)
    att = e / jnp.sum(e, axis=0, keepdims=True)
    att_exp = jnp.dot(att.reshape(K * N, H), gt,
                      preferred_element_type=jnp.float32)
    o = jnp.sum((att_exp * vp).reshape(K, N, D), axis=0)
    attn = jnp.dot(o, ow[...], preferred_element_type=jnp.float32) + ob[...]
    out_ref[0] = jnp.dot(attn, pw[...],
                         preferred_element_type=jnp.float32) + pb[...]


def kernel(x_scalar, season_q, year_q, dw_w, dw_b, pw_w, pw_b, ln_w, ln_b,
           in_proj_w, in_proj_b, out_proj_w, out_proj_b, proj_w, proj_b,
           memory_bank, memory_seasons, memory_years):
    f32 = jnp.float32
    x_scalar = x_scalar.astype(f32)
    season_q = season_q.astype(jnp.int32)
    year_q = year_q.astype(f32)
    memory_seasons = memory_seasons.astype(jnp.int32)
    memory_years = memory_years.astype(f32)

    x_pad = jnp.pad(x_scalar, ((0, 0), (6, 6), (0, 0)))
    w_t = jnp.transpose(dw_w[:, 0, :]).reshape(KW, 1, N)
    n_o = (N * D) // O_BLK
    q_pre = pl.pallas_call(
        _enc_kernel,
        grid=(n_o,),
        in_specs=[
            pl.BlockSpec((B, T + KW, N), lambda o: (0, 0, 0)),
            pl.BlockSpec((KW, 1, N), lambda o: (0, 0, 0)),
            pl.BlockSpec((1, 1, N), lambda o: (0, 0, 0)),
            pl.BlockSpec((O_BLK, N), lambda o: (o, 0)),
            pl.BlockSpec((1, O_BLK), lambda o: (0, o)),
        ],
        out_specs=pl.BlockSpec((B, O_BLK), lambda o: (0, o)),
        out_shape=jax.ShapeDtypeStruct((B, N * D), f32),
        scratch_shapes=[pltpu.VMEM((T_OUT * B, N), f32)],
    )(x_pad, w_t, dw_b.reshape(1, 1, N), pw_w, pw_b.reshape(1, N * D))

    n_m = M // M_BLK
    q3, topk_idx = pl.pallas_call(
        functools.partial(_simtop_kernel, n_m=n_m),
        grid=(n_m,),
        in_specs=[
            pl.BlockSpec((B, N, D), lambda m: (0, 0, 0)),
            pl.BlockSpec((M_BLK, N, D), lambda m: (m, 0, 0)),
            pl.BlockSpec((M_BLK, 1), lambda m: (m, 0)),
            pl.BlockSpec((M_BLK, 1), lambda m: (m, 0)),
            pl.BlockSpec((1, B), lambda m: (0, 0)),
            pl.BlockSpec((1, B), lambda m: (0, 0)),
            pl.BlockSpec((1, 1, D), lambda m: (0, 0, 0)),
            pl.BlockSpec((1, 1, D), lambda m: (0, 0, 0)),
        ],
        out_specs=[
            pl.BlockSpec((B, N, D), lambda m: (0, 0, 0)),
            pl.BlockSpec((K, B), lambda m: (0, 0)),
        ],
        out_shape=[
            jax.ShapeDtypeStruct((B, N, D), f32),
            jax.ShapeDtypeStruct((K, B), jnp.int32),
        ],
        scratch_shapes=[pltpu.VMEM((B, N * D), f32),
                        pltpu.VMEM((M, B), f32)],
    )(q_pre.reshape(B, N, D), memory_bank, memory_seasons.reshape(M, 1),
      memory_years.reshape(M, 1), season_q.reshape(1, B),
      year_q.reshape(1, B), ln_w.reshape(1, 1, D), ln_b.reshape(1, 1, D))

    wq_t = jnp.transpose(in_proj_w[:D])
    wk_t = jnp.transpose(in_proj_w[D:2 * D])
    wv_t = jnp.transpose(in_proj_w[2 * D:])
    bq = in_proj_b[:D].reshape(1, D)
    bk = in_proj_b[D:2 * D].reshape(1, D)
    bv = in_proj_b[2 * D:].reshape(1, D)
    ow_t = jnp.transpose(out_proj_w)
    pw_t = jnp.transpose(proj_w)

    mem_specs = [
        pl.BlockSpec((1, N, D), functools.partial(
            lambda b, idx, kk: (idx[kk, b], 0, 0), kk=k))
        for k in range(K)
    ]
    out = pl.pallas_call(
        _attn_kernel,
        grid_spec=pltpu.PrefetchScalarGridSpec(
            num_scalar_prefetch=1,
            grid=(B,),
            in_specs=[pl.BlockSpec((1, N, D), lambda b, idx: (b, 0, 0))]
            + mem_specs
            + [
                pl.BlockSpec((D, D), lambda b, idx: (0, 0)),
                pl.BlockSpec((D, D), lambda b, idx: (0, 0)),
                pl.BlockSpec((D, D), lambda b, idx: (0, 0)),
                pl.BlockSpec((1, D), lambda b, idx: (0, 0)),
                pl.BlockSpec((1, D), lambda b, idx: (0, 0)),
                pl.BlockSpec((1, D), lambda b, idx: (0, 0)),
                pl.BlockSpec((D, D), lambda b, idx: (0, 0)),
                pl.BlockSpec((1, D), lambda b, idx: (0, 0)),
                pl.BlockSpec((D, D), lambda b, idx: (0, 0)),
                pl.BlockSpec((1, D), lambda b, idx: (0, 0)),
            ],
            out_specs=pl.BlockSpec((1, N, D), lambda b, idx: (b, 0, 0)),
        ),
        out_shape=jax.ShapeDtypeStruct((B, N, D), f32),
    )(topk_idx, q3, *([memory_bank] * K), wq_t, wk_t, wv_t, bq, bk, bv,
      ow_t, out_proj_b.reshape(1, D), pw_t, proj_b.reshape(1, D))

    return (out, q3)
